# baseline (speedup 1.0000x reference)
import functools

import jax
import jax.numpy as jnp
from jax import lax
from jax.experimental import pallas as pl
from jax.experimental.pallas import tpu as pltpu
from jax.experimental.pallas import tpu_sc as plsc

D_IN = 1024
D_HID = 2048
N_EXP = 8
N_TOK = 4096
TOK_BLK = 256
ROW_BLK = 256
N_BLK = 40
N_SLOT = N_BLK * ROW_BLK
N_WORKER = 32
ROWS_PER_W = N_SLOT // N_WORKER
G_CHUNK = 16
G_NBUF = 4
TOK_PER_TILE = N_TOK // 16
TOK_PER_W = N_TOK // N_WORKER
C_CHUNK = 8

_NEG = float("-inf")
_DIAG_MODE = 1


def _gate_body(xb_ref, gw_ref, sc_ref, xsc_ref):
    xb = xb_ref[...]
    scores = lax.dot_general(
        xb, gw_ref[...].astype(jnp.bfloat16), (((1,), (1,)), ((), ())),
        preferred_element_type=jnp.float32,
    )
    m1 = jnp.max(scores, axis=1, keepdims=True)
    sc_ref[...] = scores.T
    xsc_ref[...] = (xb.astype(jnp.float32) * m1).reshape(TOK_BLK, 8, 128)


def _rg_body(sc_hbm, xsc_hbm, xs_out, p0_out, p1_out, nbp_out,
             scs_v, a1_v, a2_v, bins_v, binsall_v, cnt_v, nbp_v,
             p0l_v, p1l_v, dest_v, tokv_v, stl_v, zbuf_v,
             gbuf0, gbuf1, gbuf2, gbuf3, ib0, ib1, ib2, ib3, bins_sh, st_sh,
             gsem0, gsem1, gsem2, gsem3, ssem0, ssem1, ssem2, ssem3):
    cid = lax.axis_index("c")
    sid = lax.axis_index("s")
    wid = sid * 2 + cid
    lane = lax.iota(jnp.int32, 16)
    ones = jnp.ones((16,), jnp.int32)
    zeros_i = jnp.zeros((16,), jnp.int32)

    t0 = sid * TOK_PER_TILE
    for j in range(N_EXP):
        pltpu.sync_copy(sc_hbm.at[j, pl.ds(t0, TOK_PER_TILE)], scs_v.at[j])
    for e in range(N_EXP):
        bins_v[pl.ds(e * 16, 16)] = zeros_i
    for q in range(40):
        zbuf_v[pl.ds(q * 16, 16)] = zeros_i

    for k in range(TOK_PER_TILE // 16):
        vs = [scs_v[j, pl.ds(k * 16, 16)] for j in range(N_EXP)]
        m1 = functools.reduce(jnp.maximum, vs)
        a1 = functools.reduce(
            jnp.minimum,
            [jnp.where(vs[j] == m1, j, N_EXP) for j in range(N_EXP)])
        vs2 = [jnp.where(a1 == j, _NEG, vs[j]) for j in range(N_EXP)]
        m2 = functools.reduce(jnp.maximum, vs2)
        a2 = functools.reduce(
            jnp.minimum,
            [jnp.where(vs2[j] == m2, j, N_EXP) for j in range(N_EXP)])
        a1_v[pl.ds(k * 16, 16)] = a1
        a2_v[pl.ds(k * 16, 16)] = a2
        plsc.addupdate_scatter(bins_v, [a1 * 16 + lane], ones)
        plsc.addupdate_scatter(bins_v, [a2 * 16 + lane], ones)

    pltpu.sync_copy(bins_v, bins_sh.at[sid])
    pltpu.sync_copy(zbuf_v, st_sh.at[pl.ds(sid * (N_SLOT // 16), N_SLOT // 16)])
    plsc.subcore_barrier()

    pltpu.sync_copy(bins_sh, binsall_v)
    base = jnp.int32(0)
    nbp_vec = jnp.zeros((16,), jnp.int32)
    for e in range(N_EXP):
        nbp_vec = jnp.where(lane == e, base, nbp_vec)
        run = base
        mybase = zeros_i
        for sp in range(16):
            row = binsall_v[sp, pl.ds(e * 16, 16)]
            cum = plsc.cumsum(row)
            mybase = jnp.where(sid == sp, run + (cum - row), mybase)
            run = run + jnp.sum(row)
        cnt_v[pl.ds(e * 16, 16)] = mybase
        base = base + ((run - base + (ROW_BLK - 1)) >> 8 << 8)
    nbp_vec = jnp.where(lane >= N_EXP, base, nbp_vec)

    for k in range(TOK_PER_TILE // 16):
        tok = t0 + k * 16 + lane
        a1 = a1_v[pl.ds(k * 16, 16)]
        i1 = a1 * 16 + lane
        d1 = plsc.load_gather(cnt_v, [i1])
        plsc.store_scatter(cnt_v, [i1], d1 + 1)
        p0l_v[pl.ds(k * 16, 16)] = d1
        dest_v[k // 8, pl.ds((k % 8) * 16, 16)] = d1
        tokv_v[k // 8, pl.ds((k % 8) * 16, 16)] = tok
        a2 = a2_v[pl.ds(k * 16, 16)]
        i2 = a2 * 16 + lane
        d2 = plsc.load_gather(cnt_v, [i2])
        plsc.store_scatter(cnt_v, [i2], d2 + 1)
        p1l_v[pl.ds(k * 16, 16)] = d2
        dest_v[2 + k // 8, pl.ds((k % 8) * 16, 16)] = d2
        tokv_v[2 + k // 8, pl.ds((k % 8) * 16, 16)] = tok

    for j in range(4):
        pltpu.sync_copy(tokv_v.at[j], st_sh.at[dest_v.at[j]])

    @pl.when(cid == 0)
    def _():
        pltpu.sync_copy(p0l_v, p0_out.at[pl.ds(t0, TOK_PER_TILE)])
        pltpu.sync_copy(p1l_v, p1_out.at[pl.ds(t0, TOK_PER_TILE)])

    @pl.when(jnp.logical_and(cid == 0, sid == 0))
    def _():
        nbp_v[...] = nbp_vec
        pltpu.sync_copy(nbp_v, nbp_out)

    plsc.subcore_barrier()

    pltpu.sync_copy(st_sh.at[pl.ds(wid * ROWS_PER_W, ROWS_PER_W)], stl_v)
    n_chunk = ROWS_PER_W // G_CHUNK
    gb = [gbuf0, gbuf1, gbuf2, gbuf3]
    gsem = [gsem0, gsem1, gsem2, gsem3]
    ssem = [ssem0, ssem1, ssem2, ssem3]

    ib = [ib0, ib1, ib2, ib3]

    def _start(c):
        ib[c % G_NBUF][...] = stl_v[pl.ds(c * G_CHUNK, G_CHUNK)]
        return pltpu.async_copy(
            xsc_hbm.at[ib[c % G_NBUF]], gb[c % G_NBUF], gsem[c % G_NBUF])

    if _DIAG_MODE == 1:
        gcp = {c: _start(c) for c in range(G_NBUF)}
        for c in range(n_chunk):
            gcp[c].wait()
            if c + G_NBUF < n_chunk:
                gcp[c + G_NBUF] = _start(c + G_NBUF)
        return
    if _DIAG_MODE == 2:
        scp = {}
        for c in range(n_chunk):
            s = wid * ROWS_PER_W + c * G_CHUNK
            scp[c] = pltpu.async_copy(
                gb[c % G_NBUF], xs_out.at[pl.ds(s, G_CHUNK)], ssem[c % G_NBUF])
            if c + G_NBUF < n_chunk:
                scp[c].wait()
        for c in range(max(0, n_chunk - G_NBUF), n_chunk):
            scp[c].wait()
        return
    gcp = {c: _start(c) for c in range(G_NBUF)}
    scp = {}
    for c in range(n_chunk):
        gcp[c].wait()
        s = wid * ROWS_PER_W + c * G_CHUNK
        scp[c] = pltpu.async_copy(
            gb[c % G_NBUF], xs_out.at[pl.ds(s, G_CHUNK)], ssem[c % G_NBUF])
        if c + G_NBUF < n_chunk:
            scp[c].wait()
            gcp[c + G_NBUF] = _start(c + G_NBUF)
    for c in range(max(0, n_chunk - G_NBUF), n_chunk):
        scp[c].wait()


def _exp_body(nbp_ref, xs_ref, ew_ref, hs_ref):
    b = pl.program_id(0)
    row0 = b * ROW_BLK
    e = jnp.int32(0)
    for j in range(1, N_EXP):
        e = e + jnp.where(row0 >= nbp_ref[j], 1, 0)
    w = ew_ref[e]
    xs = xs_ref[...].reshape(ROW_BLK, D_IN)
    h = lax.dot_general(
        xs.astype(jnp.bfloat16), w, (((1,), (1,)), ((), ())),
        preferred_element_type=jnp.float32,
    )
    hs_ref[...] = h


def _comb_body(hs_hbm, p0_hbm, p1_hbm, h0_out, h1_out,
               p0_v, p1_v, b0a, b0b, b1a, b1b,
               g0sem, g1sem, s0sem, s1sem):
    cid = lax.axis_index("c")
    sid = lax.axis_index("s")
    wid = sid * 2 + cid
    tbase = wid * TOK_PER_W
    pltpu.sync_copy(p0_hbm.at[pl.ds(tbase, TOK_PER_W)], p0_v)
    pltpu.sync_copy(p1_hbm.at[pl.ds(tbase, TOK_PER_W)], p1_v)
    n_chunk = TOK_PER_W // C_CHUNK
    b0 = [b0a, b0b]
    b1 = [b1a, b1b]

    def _start(c):
        sl = pl.ds(c * C_CHUNK, C_CHUNK)
        return (
            pltpu.async_copy(hs_hbm.at[p0_v.at[sl]], b0[c % 2], g0sem),
            pltpu.async_copy(hs_hbm.at[p1_v.at[sl]], b1[c % 2], g1sem),
        )

    gcp = {0: _start(0), 1: _start(1)}
    scp = {}
    for c in range(n_chunk):
        gcp[c][0].wait()
        gcp[c][1].wait()
        sl = pl.ds(tbase + c * C_CHUNK, C_CHUNK)
        scp[c] = (
            pltpu.async_copy(b0[c % 2], h0_out.at[sl], s0sem),
            pltpu.async_copy(b1[c % 2], h1_out.at[sl], s1sem),
        )
        if c + 2 < n_chunk:
            scp[c][0].wait()
            scp[c][1].wait()
            gcp[c + 2] = _start(c + 2)
    for c in (n_chunk - 2, n_chunk - 1):
        scp[c][0].wait()
        scp[c][1].wait()


def _out_body(h0_ref, h1_ref, ow_ref, o_ref):
    f = h0_ref[...] + h1_ref[...]
    g = jnp.square(jnp.maximum(f, 0.0)).astype(jnp.bfloat16)
    o_ref[...] = lax.dot_general(
        g, ow_ref[...], (((1,), (1,)), ((), ())),
        preferred_element_type=jnp.float32,
    )


@jax.jit
def kernel(x, expert_w, gate_w, out_w):
    bsz, seql, _ = x.shape
    xf = x.reshape(N_TOK, D_IN)
    xb = xf.astype(jnp.bfloat16)
    ew = expert_w.astype(jnp.bfloat16)
    ow = out_w.astype(jnp.bfloat16)

    scores, xsc = pl.pallas_call(
        _gate_body,
        grid=(N_TOK // TOK_BLK,),
        in_specs=[
            pl.BlockSpec((TOK_BLK, D_IN), lambda i: (i, 0)),
            pl.BlockSpec((N_EXP, D_IN), lambda i: (0, 0)),
        ],
        out_specs=[
            pl.BlockSpec((N_EXP, TOK_BLK), lambda i: (0, i)),
            pl.BlockSpec((TOK_BLK, 8, 128), lambda i: (i, 0, 0)),
        ],
        out_shape=[
            jax.ShapeDtypeStruct((N_EXP, N_TOK), jnp.float32),
            jax.ShapeDtypeStruct((N_TOK, 8, 128), jnp.float32),
        ],
    )(xb, gate_w)

    mesh = plsc.VectorSubcoreMesh(core_axis_name="c", subcore_axis_name="s")
    xs, p0, p1, nbp = pl.kernel(
        _rg_body,
        out_type=[
            jax.ShapeDtypeStruct((N_SLOT, 8, 128), jnp.float32),
            jax.ShapeDtypeStruct((N_TOK,), jnp.int32),
            jax.ShapeDtypeStruct((N_TOK,), jnp.int32),
            jax.ShapeDtypeStruct((16,), jnp.int32),
        ],
        mesh=mesh,
        compiler_params=pltpu.CompilerParams(needs_layout_passes=False),
        scratch_types=[
            pltpu.VMEM((N_EXP, TOK_PER_TILE), jnp.float32),
            pltpu.VMEM((TOK_PER_TILE,), jnp.int32),
            pltpu.VMEM((TOK_PER_TILE,), jnp.int32),
            pltpu.VMEM((128,), jnp.int32),
            pltpu.VMEM((16, 128), jnp.int32),
            pltpu.VMEM((128,), jnp.int32),
            pltpu.VMEM((16,), jnp.int32),
            pltpu.VMEM((TOK_PER_TILE,), jnp.int32),
            pltpu.VMEM((TOK_PER_TILE,), jnp.int32),
            pltpu.VMEM((4, 128), jnp.int32),
            pltpu.VMEM((4, 128), jnp.int32),
            pltpu.VMEM((ROWS_PER_W,), jnp.int32),
            pltpu.VMEM((N_SLOT // 16,), jnp.int32),
            pltpu.VMEM((G_CHUNK, 8, 128), jnp.float32),
            pltpu.VMEM((G_CHUNK, 8, 128), jnp.float32),
            pltpu.VMEM((G_CHUNK, 8, 128), jnp.float32),
            pltpu.VMEM((G_CHUNK, 8, 128), jnp.float32),
            pltpu.VMEM((G_CHUNK,), jnp.int32),
            pltpu.VMEM((G_CHUNK,), jnp.int32),
            pltpu.VMEM((G_CHUNK,), jnp.int32),
            pltpu.VMEM((G_CHUNK,), jnp.int32),
            pltpu.VMEM_SHARED((16, 128), jnp.int32),
            pltpu.VMEM_SHARED((N_SLOT,), jnp.int32),
            pltpu.SemaphoreType.DMA,
            pltpu.SemaphoreType.DMA,
            pltpu.SemaphoreType.DMA,
            pltpu.SemaphoreType.DMA,
            pltpu.SemaphoreType.DMA,
            pltpu.SemaphoreType.DMA,
            pltpu.SemaphoreType.DMA,
            pltpu.SemaphoreType.DMA,
        ],
    )(scores, xsc)

    hs = pl.pallas_call(
        _exp_body,
        grid=(N_BLK,),
        in_specs=[
            pl.BlockSpec(memory_space=pltpu.SMEM),
            pl.BlockSpec((ROW_BLK, 8, 128), lambda i: (i, 0, 0)),
            pl.BlockSpec((N_EXP, D_HID, D_IN), lambda i: (0, 0, 0)),
        ],
        out_specs=pl.BlockSpec((ROW_BLK, D_HID), lambda i: (i, 0)),
        out_shape=jax.ShapeDtypeStruct((N_SLOT, D_HID), jnp.float32),
    )(nbp, xs, ew)

    h0, h1 = pl.kernel(
        _comb_body,
        out_type=[
            jax.ShapeDtypeStruct((N_TOK, D_HID), jnp.float32),
            jax.ShapeDtypeStruct((N_TOK, D_HID), jnp.float32),
        ],
        mesh=mesh,
        compiler_params=pltpu.CompilerParams(needs_layout_passes=False),
        scratch_types=[
            pltpu.VMEM((TOK_PER_W,), jnp.int32),
            pltpu.VMEM((TOK_PER_W,), jnp.int32),
            pltpu.VMEM((C_CHUNK, D_HID), jnp.float32),
            pltpu.VMEM((C_CHUNK, D_HID), jnp.float32),
            pltpu.VMEM((C_CHUNK, D_HID), jnp.float32),
            pltpu.VMEM((C_CHUNK, D_HID), jnp.float32),
            pltpu.SemaphoreType.DMA,
            pltpu.SemaphoreType.DMA,
            pltpu.SemaphoreType.DMA,
            pltpu.SemaphoreType.DMA,
        ],
    )(hs, p0, p1)

    out = pl.pallas_call(
        _out_body,
        grid=(N_TOK // 512,),
        in_specs=[
            pl.BlockSpec((512, D_HID), lambda i: (i, 0)),
            pl.BlockSpec((512, D_HID), lambda i: (i, 0)),
            pl.BlockSpec((D_IN, D_HID), lambda i: (0, 0)),
        ],
        out_specs=pl.BlockSpec((512, D_IN), lambda i: (i, 0)),
        out_shape=jax.ShapeDtypeStruct((N_TOK, D_IN), jnp.float32),
    )(h0, h1, ow)
    return out.reshape(bsz, seql, D_IN)


# speedup vs baseline: 1.6165x; 1.6165x over previous
import functools

import jax
import jax.numpy as jnp
from jax.experimental import pallas as pl


INPUT_DIM = 1024
INTER_DIM = 2048
GATE_NUM = 8
TOP_K = 2

TOK_BLK = 512


def _body(x_ref, ew_ref, gw_ref, ow_ref, o_ref):
    x = x_ref[...]
    xb = x.astype(jnp.bfloat16)
    scores = jax.lax.dot_general(
        xb, gw_ref[...].astype(jnp.bfloat16), (((1,), (1,)), ((), ())),
        preferred_element_type=jnp.float32,
    )
    idx = jax.lax.broadcasted_iota(jnp.int32, scores.shape, 1)
    m1 = jnp.max(scores, axis=1, keepdims=True)
    a1 = jnp.min(jnp.where(scores == m1, idx, GATE_NUM), axis=1, keepdims=True)
    scores2 = jnp.where(idx == a1, -jnp.inf, scores)
    m2 = jnp.max(scores2, axis=1, keepdims=True)
    a2 = jnp.min(jnp.where(scores2 == m2, idx, GATE_NUM), axis=1, keepdims=True)

    acc = jnp.zeros((x.shape[0], INTER_DIM), jnp.float32)
    for e in range(GATE_NUM):
        sel = ((a1 == e) | (a2 == e)).astype(jnp.float32)
        h = jax.lax.dot_general(
            xb, ew_ref[e], (((1,), (1,)), ((), ())),
            preferred_element_type=jnp.float32,
        )
        acc = acc + h * (m1 * sel)
    g = jnp.square(jnp.maximum(acc, 0.0)).astype(jnp.bfloat16)
    o_ref[...] = jax.lax.dot_general(
        g, ow_ref[...], (((1,), (1,)), ((), ())),
        preferred_element_type=jnp.float32,
    )


@jax.jit
def kernel(x, expert_w, gate_w, out_w):
    bsz, seql, _ = x.shape
    n_tok = bsz * seql
    xf = x.reshape(n_tok, INPUT_DIM)
    ew = expert_w.astype(jnp.bfloat16)
    ow = out_w.astype(jnp.bfloat16)
    out = pl.pallas_call(
        _body,
        grid=(n_tok // TOK_BLK,),
        in_specs=[
            pl.BlockSpec((TOK_BLK, INPUT_DIM), lambda i: (i, 0)),
            pl.BlockSpec((GATE_NUM, INTER_DIM, INPUT_DIM), lambda i: (0, 0, 0)),
            pl.BlockSpec((GATE_NUM, INPUT_DIM), lambda i: (0, 0)),
            pl.BlockSpec((INPUT_DIM, INTER_DIM), lambda i: (0, 0)),
        ],
        out_specs=pl.BlockSpec((TOK_BLK, INPUT_DIM), lambda i: (i, 0)),
        out_shape=jax.ShapeDtypeStruct((n_tok, INPUT_DIM), jnp.float32),
    )(xf, ew, gate_w, ow)
    return out.reshape(bsz, seql, INPUT_DIM)
